# baseline (speedup 1.0000x reference)
import jax
import jax.numpy as jnp
from jax.experimental import pallas as pl
from jax.experimental.pallas import tpu as pltpu


N_BLK = 2


def _mm_kernel(wsel_ref, osel_ref, x_ref, w_ref, b_ref, o_ref):
    del wsel_ref
    p = pl.program_id(0)
    i = osel_ref[p] // 2
    acc = jax.lax.dot_general(
        x_ref[i], w_ref[0],
        dimension_numbers=(((1,), (1,)), ((), ())),
        preferred_element_type=jnp.float32,
    )
    o_ref[...] = (acc + b_ref[0])[None]


def kernel(x, indices, weight, bias):
    G, S = indices.shape
    E, OUT_D, IN_D = weight.shape
    T = x.shape[-2]
    P = G * S
    OB = OUT_D // N_BLK

    idx = indices.reshape(P)
    order = jnp.argsort(idx)
    wsel = jnp.take(idx, order)

    xr = x.reshape(G, T, IN_D)
    br = bias.reshape(E, 1, OUT_D)

    grid_spec = pltpu.PrefetchScalarGridSpec(
        num_scalar_prefetch=2,
        grid=(P, N_BLK),
        in_specs=[
            pl.BlockSpec((G, T, IN_D),
                         lambda p, n, wsel, osel: (0, 0, 0)),
            pl.BlockSpec((1, OB, IN_D),
                         lambda p, n, wsel, osel: (wsel[p], n, 0)),
            pl.BlockSpec((1, 1, OB),
                         lambda p, n, wsel, osel: (wsel[p], 0, n)),
        ],
        out_specs=pl.BlockSpec((1, T, OB),
                               lambda p, n, wsel, osel: (osel[p], 0, n)),
    )

    out = pl.pallas_call(
        _mm_kernel,
        grid_spec=grid_spec,
        out_shape=jax.ShapeDtypeStruct((P, T, OUT_D), jnp.float32),
    )(wsel, order, xr, weight, br)

    return out.reshape(1, G, S, T, OUT_D)


# speedup vs baseline: 1.4500x; 1.4500x over previous
import jax
import jax.numpy as jnp
from jax.experimental import pallas as pl
from jax.experimental.pallas import tpu as pltpu


N_BLK = 1


def _mm_kernel(wsel_ref, osel_ref, x_ref, w_ref, b_ref, o_ref):
    del wsel_ref
    p = pl.program_id(0)
    i = osel_ref[p] // 2
    acc = jax.lax.dot_general(
        x_ref[i], w_ref[0],
        dimension_numbers=(((1,), (1,)), ((), ())),
        preferred_element_type=jnp.float32,
    )
    o_ref[...] = (acc + b_ref[0])[None]


def kernel(x, indices, weight, bias):
    G, S = indices.shape
    E, OUT_D, IN_D = weight.shape
    T = x.shape[-2]
    P = G * S
    OB = OUT_D // N_BLK

    idx = indices.reshape(P)
    order = jnp.argsort(idx)
    wsel = jnp.take(idx, order)

    xr = x.reshape(G, T, IN_D)
    br = bias.reshape(E, 1, OUT_D)

    grid_spec = pltpu.PrefetchScalarGridSpec(
        num_scalar_prefetch=2,
        grid=(P, N_BLK),
        in_specs=[
            pl.BlockSpec((G, T, IN_D),
                         lambda p, n, wsel, osel: (0, 0, 0)),
            pl.BlockSpec((1, OB, IN_D),
                         lambda p, n, wsel, osel: (wsel[p], n, 0)),
            pl.BlockSpec((1, 1, OB),
                         lambda p, n, wsel, osel: (wsel[p], 0, n)),
        ],
        out_specs=pl.BlockSpec((1, T, OB),
                               lambda p, n, wsel, osel: (osel[p], 0, n)),
    )

    out = pl.pallas_call(
        _mm_kernel,
        grid_spec=grid_spec,
        out_shape=jax.ShapeDtypeStruct((P, T, OUT_D), jnp.float32),
    )(wsel, order, xr, weight, br)

    return out.reshape(1, G, S, T, OUT_D)


# speedup vs baseline: 1.4570x; 1.0048x over previous
import jax
import jax.numpy as jnp
from jax.experimental import pallas as pl
from jax.experimental.pallas import tpu as pltpu


W_SPLIT = 2


def _mm_kernel(wsel_ref, osel_ref, x_ref, *rest):
    del wsel_ref
    w_refs = rest[:-2]
    b_ref, o_ref = rest[-2:]
    p = pl.program_id(0)
    i = osel_ref[p] // 2
    xa = x_ref[i]
    ob = o_ref.shape[-1] // len(w_refs)
    for s, w_ref in enumerate(w_refs):
        acc = jax.lax.dot_general(
            xa, w_ref[0, 0],
            dimension_numbers=(((1,), (1,)), ((), ())),
            preferred_element_type=jnp.float32,
        )
        o_ref[0, :, s * ob:(s + 1) * ob] = acc + b_ref[0, 0, s * ob:(s + 1) * ob]


def kernel(x, indices, weight, bias):
    G, S = indices.shape
    E, OUT_D, IN_D = weight.shape
    T = x.shape[-2]
    P = G * S
    OB = OUT_D // W_SPLIT

    idx = indices.reshape(P)
    order = jnp.argsort(idx)
    wsel = jnp.take(idx, order)

    xr = x.reshape(G, T, IN_D)
    ws = weight.reshape(E, W_SPLIT, OB, IN_D)
    br = bias.reshape(E, 1, OUT_D)

    def _wmap(s):
        return lambda p, wsel, osel: (wsel[p], s, 0, 0)

    grid_spec = pltpu.PrefetchScalarGridSpec(
        num_scalar_prefetch=2,
        grid=(P,),
        in_specs=[
            pl.BlockSpec((G, T, IN_D),
                         lambda p, wsel, osel: (0, 0, 0)),
        ] + [
            pl.BlockSpec((1, 1, OB, IN_D), _wmap(s)) for s in range(W_SPLIT)
        ] + [
            pl.BlockSpec((1, 1, OUT_D),
                         lambda p, wsel, osel: (wsel[p], 0, 0)),
        ],
        out_specs=pl.BlockSpec((1, T, OUT_D),
                               lambda p, wsel, osel: (osel[p], 0, 0)),
    )

    out = pl.pallas_call(
        _mm_kernel,
        grid_spec=grid_spec,
        out_shape=jax.ShapeDtypeStruct((P, T, OUT_D), jnp.float32),
    )(wsel, order, xr, *([ws] * W_SPLIT), br)

    return out.reshape(1, G, S, T, OUT_D)


# speedup vs baseline: 1.5389x; 1.0562x over previous
import jax
import jax.numpy as jnp
from jax.experimental import pallas as pl
from jax.experimental.pallas import tpu as pltpu


def _mm_kernel(idx_ref, x_ref, *rest):
    del idx_ref
    n = (len(rest) - 1) // 2
    w_refs = rest[:n]
    b_refs = rest[n:2 * n]
    o_ref = rest[-1]
    i = pl.program_id(0)
    xa = x_ref[i]
    for s in range(n):
        acc = jax.lax.dot_general(
            xa, w_refs[s][0],
            dimension_numbers=(((1,), (1,)), ((), ())),
            preferred_element_type=jnp.float32,
        )
        o_ref[0, s] = acc + b_refs[s][0]


def kernel(x, indices, weight, bias):
    G, S = indices.shape
    E, OUT_D, IN_D = weight.shape
    T = x.shape[-2]

    xr = x.reshape(G, T, IN_D)
    br = bias.reshape(E, 1, OUT_D)

    def _wmap(s):
        return lambda i, ind: (ind[i, s], 0, 0)

    grid_spec = pltpu.PrefetchScalarGridSpec(
        num_scalar_prefetch=1,
        grid=(G,),
        in_specs=[
            pl.BlockSpec((G, T, IN_D), lambda i, ind: (0, 0, 0)),
        ] + [
            pl.BlockSpec((1, OUT_D, IN_D), _wmap(s)) for s in range(S)
        ] + [
            pl.BlockSpec((1, 1, OUT_D), _wmap(s)) for s in range(S)
        ],
        out_specs=pl.BlockSpec((1, S, T, OUT_D),
                               lambda i, ind: (i, 0, 0, 0)),
    )

    out = pl.pallas_call(
        _mm_kernel,
        grid_spec=grid_spec,
        out_shape=jax.ShapeDtypeStruct((G, S, T, OUT_D), jnp.float32),
    )(indices, xr, *([weight] * S), *([br] * S))

    return out.reshape(1, G, S, T, OUT_D)
